# Initial kernel scaffold; baseline (speedup 1.0000x reference)
import jax
import jax.numpy as jnp
from jax.experimental import pallas as pl


def kernel(loc_data, conf_data, obj_data, priors, targets):
    raise NotImplementedError("write your pallas kernel here")


# speedup vs baseline: 65.9652x; 65.9652x over previous
import functools

import jax
import jax.numpy as jnp
from jax import lax
from jax.experimental import pallas as pl
from jax.experimental.pallas import tpu as pltpu

_NUM_CLASSES = 20
_THRESHOLD = 0.5
_NEGPOS_RATIO = 3
_VAR0 = 0.1
_VAR1 = 0.2
_BATCH = 32
_NUM_PRIORS = 32768
_NUM_OBJS = 8
_R = _NUM_PRIORS // 128


def _topk_sum(vals, k):
    keys = lax.bitcast_convert_type(vals, jnp.int32)
    lo = jnp.int32(0)
    for bit in range(30, -1, -1):
        cand = jnp.bitwise_or(lo, jnp.int32(1 << bit))
        cnt = jnp.sum((keys >= cand).astype(jnp.int32))
        lo = jnp.where(cnt >= k, cand, lo)
    gt = keys > lo
    cnt_gt = jnp.sum(gt.astype(jnp.int32))
    sum_gt = jnp.sum(jnp.where(gt, vals, 0.0))
    t_f = lax.bitcast_convert_type(lo, jnp.float32)
    corr = jnp.where(k > 0, (k - cnt_gt).astype(jnp.float32) * t_f, 0.0)
    return sum_gt + corr


def _mbox_kernel(priors_ref, targets_ref, loc_ref, conf_ref, obj_ref, out_ref):
    b = pl.program_id(0)

    pcx = priors_ref[0]
    pcy = priors_ref[1]
    pw = priors_ref[2]
    ph = priors_ref[3]
    px1 = pcx - pw * 0.5
    py1 = pcy - ph * 0.5
    px2 = pcx + pw * 0.5
    py2 = pcy + ph * 0.5
    area_p = (px2 - px1) * (py2 - py1)

    fi = (lax.broadcasted_iota(jnp.int32, (_R, 128), 0) * 128
          + lax.broadcasted_iota(jnp.int32, (_R, 128), 1))

    tx1 = [targets_ref[0, j, 0] for j in range(_NUM_OBJS)]
    ty1 = [targets_ref[0, j, 1] for j in range(_NUM_OBJS)]
    tx2 = [targets_ref[0, j, 2] for j in range(_NUM_OBJS)]
    ty2 = [targets_ref[0, j, 3] for j in range(_NUM_OBJS)]
    tlab = [targets_ref[0, j, 4] for j in range(_NUM_OBJS)]

    bto = None
    bti = None
    bpi = []
    for j in range(_NUM_OBJS):
        ix = jnp.maximum(jnp.minimum(tx2[j], px2) - jnp.maximum(tx1[j], px1), 0.0)
        iy = jnp.maximum(jnp.minimum(ty2[j], py2) - jnp.maximum(ty1[j], py1), 0.0)
        inter = ix * iy
        area_t = (tx2[j] - tx1[j]) * (ty2[j] - ty1[j])
        ov = inter / (area_t + area_p - inter)
        if j == 0:
            bto = ov
            bti = jnp.zeros((_R, 128), jnp.int32)
        else:
            upd = ov > bto
            bti = jnp.where(upd, jnp.int32(j), bti)
            bto = jnp.where(upd, ov, bto)
        m = jnp.max(ov)
        bpi.append(jnp.min(jnp.where(ov == m, fi, jnp.int32(2 ** 30))))

    for j in range(_NUM_OBJS):
        hit = fi == bpi[j]
        bto = jnp.where(hit, 2.0, bto)
        bti = jnp.where(hit, jnp.int32(j), bti)

    conf_t = jnp.where(bti == 0, tlab[0].astype(jnp.int32), 0)
    mx1 = jnp.where(bti == 0, tx1[0], 0.0)
    my1 = jnp.where(bti == 0, ty1[0], 0.0)
    mx2 = jnp.where(bti == 0, tx2[0], 0.0)
    my2 = jnp.where(bti == 0, ty2[0], 0.0)
    for j in range(1, _NUM_OBJS):
        sel = bti == j
        conf_t = jnp.where(sel, tlab[j].astype(jnp.int32), conf_t)
        mx1 = jnp.where(sel, tx1[j], mx1)
        my1 = jnp.where(sel, ty1[j], my1)
        mx2 = jnp.where(sel, tx2[j], mx2)
        my2 = jnp.where(sel, ty2[j], my2)
    conf_t = jnp.where(bto < _THRESHOLD, 0, conf_t)
    pos = conf_t > 0
    posf = pos.astype(jnp.float32)
    num_pos = jnp.sum(conf_t > 0, dtype=jnp.int32)

    g_cx = ((mx1 + mx2) * 0.5 - pcx) / (_VAR0 * pw)
    g_cy = ((my1 + my2) * 0.5 - pcy) / (_VAR0 * ph)
    g_w = jnp.log((mx2 - mx1) / pw) / _VAR1
    g_h = jnp.log((my2 - my1) / ph) / _VAR1
    loss_l = jnp.float32(0.0)
    for c, g in enumerate((g_cx, g_cy, g_w, g_h)):
        d = loc_ref[0, c] - g
        ad = jnp.abs(d)
        sl1 = jnp.where(ad < 1.0, 0.5 * d * d, ad - 0.5)
        loss_l = loss_l + jnp.sum(sl1 * posf)

    o0 = obj_ref[0, 0]
    o1 = obj_ref[0, 1]
    mo = jnp.maximum(o0, o1)
    lse_o = mo + jnp.log(jnp.exp(o0 - mo) + jnp.exp(o1 - mo))
    ce_obj = lse_o - jnp.where(pos, o1, o0)

    mc = conf_ref[0, 0]
    for c in range(1, _NUM_CLASSES):
        mc = jnp.maximum(mc, conf_ref[0, c])
    s = jnp.exp(conf_ref[0, 0] - mc)
    for c in range(1, _NUM_CLASSES):
        s = s + jnp.exp(conf_ref[0, c] - mc)
    lse_c = mc + jnp.log(s)
    c_sel = jnp.zeros((_R, 128), jnp.float32)
    for c in range(_NUM_CLASSES):
        c_sel = jnp.where(conf_t == c + 1, conf_ref[0, c], c_sel)
    ce_all = jnp.where(pos, lse_c + lse_o - o1 - c_sel, ce_obj)

    k = jnp.minimum(jnp.int32(_NEGPOS_RATIO) * num_pos,
                    jnp.int32(_NUM_PRIORS - 1))
    mined_c = jnp.where(pos, 0.0, ce_all)
    mined_o = jnp.where(pos, 0.0, ce_obj)
    loss_c = jnp.sum(jnp.where(pos, ce_all, 0.0)) + _topk_sum(mined_c, k)
    loss_o = jnp.sum(jnp.where(pos, ce_obj, 0.0)) + _topk_sum(mined_o, k)

    lane = lax.broadcasted_iota(jnp.int32, (1, 8), 1)
    row = (jnp.where(lane == 0, loss_l, 0.0)
           + jnp.where(lane == 1, loss_c, 0.0)
           + jnp.where(lane == 2, loss_o, 0.0)
           + jnp.where(lane == 3, num_pos.astype(jnp.float32), 0.0))

    @pl.when(b == 0)
    def _init():
        out_ref[...] = row

    @pl.when(b > 0)
    def _acc():
        out_ref[...] = out_ref[...] + row


@jax.jit
def _run(loc_s, conf_s, obj_s, priors_s, targets):
    out = pl.pallas_call(
        _mbox_kernel,
        grid=(_BATCH,),
        in_specs=[
            pl.BlockSpec((4, _R, 128), lambda b: (0, 0, 0)),
            pl.BlockSpec((1, _NUM_OBJS, 5), lambda b: (b, 0, 0)),
            pl.BlockSpec((1, 4, _R, 128), lambda b: (b, 0, 0, 0)),
            pl.BlockSpec((1, _NUM_CLASSES, _R, 128), lambda b: (b, 0, 0, 0)),
            pl.BlockSpec((1, 2, _R, 128), lambda b: (b, 0, 0, 0)),
        ],
        out_specs=pl.BlockSpec((1, 8), lambda b: (0, 0)),
        out_shape=jax.ShapeDtypeStruct((1, 8), jnp.float32),
        compiler_params=pltpu.CompilerParams(
            dimension_semantics=("arbitrary",),
        ),
    )(priors_s, targets, loc_s, conf_s, obj_s)
    n = out[0, 3]
    return out[0, 0] / n, out[0, 1] / n, out[0, 2] / n


def kernel(loc_data, conf_data, obj_data, priors, targets):
    loc_s = loc_data.transpose(0, 2, 1).reshape(_BATCH, 4, _R, 128)
    conf_s = conf_data.transpose(0, 2, 1).reshape(_BATCH, _NUM_CLASSES, _R, 128)
    obj_s = obj_data.transpose(0, 2, 1).reshape(_BATCH, 2, _R, 128)
    priors_s = priors.T.reshape(4, _R, 128)
    return _run(loc_s, conf_s, obj_s, priors_s, targets)


# speedup vs baseline: 69.0758x; 1.0472x over previous
import functools

import jax
import jax.numpy as jnp
from jax import lax
from jax.experimental import pallas as pl
from jax.experimental.pallas import tpu as pltpu

_NUM_CLASSES = 20
_THRESHOLD = 0.5
_NEGPOS_RATIO = 3
_VAR0 = 0.1
_VAR1 = 0.2
_BATCH = 32
_NUM_PRIORS = 32768
_NUM_OBJS = 8
_R = _NUM_PRIORS // 128


def _topk_sum(vals, k):
    keys = lax.bitcast_convert_type(vals, jnp.int32)
    lo = jnp.int32(0)
    for bit in range(30, -1, -1):
        cand = jnp.bitwise_or(lo, jnp.int32(1 << bit))
        cnt = jnp.sum((keys >= cand).astype(jnp.int32))
        lo = jnp.where(cnt >= k, cand, lo)
    gt = keys > lo
    cnt_gt = jnp.sum(gt.astype(jnp.int32))
    sum_gt = jnp.sum(jnp.where(gt, vals, 0.0))
    t_f = lax.bitcast_convert_type(lo, jnp.float32)
    corr = jnp.where(k > 0, (k - cnt_gt).astype(jnp.float32) * t_f, 0.0)
    return sum_gt + corr


def _mbox_kernel(priors_ref, targets_ref, loc_ref, conf_ref, obj_ref, out_ref):
    b = pl.program_id(0)

    pcx = priors_ref[0]
    pcy = priors_ref[1]
    pw = priors_ref[2]
    ph = priors_ref[3]
    px1 = pcx - pw * 0.5
    py1 = pcy - ph * 0.5
    px2 = pcx + pw * 0.5
    py2 = pcy + ph * 0.5
    area_p = (px2 - px1) * (py2 - py1)

    fi = (lax.broadcasted_iota(jnp.int32, (_R, 128), 0) * 128
          + lax.broadcasted_iota(jnp.int32, (_R, 128), 1))

    tx1 = [targets_ref[0, j, 0] for j in range(_NUM_OBJS)]
    ty1 = [targets_ref[0, j, 1] for j in range(_NUM_OBJS)]
    tx2 = [targets_ref[0, j, 2] for j in range(_NUM_OBJS)]
    ty2 = [targets_ref[0, j, 3] for j in range(_NUM_OBJS)]
    tlab = [targets_ref[0, j, 4] for j in range(_NUM_OBJS)]

    bto = None
    bti = None
    bpi = []
    for j in range(_NUM_OBJS):
        ix = jnp.maximum(jnp.minimum(tx2[j], px2) - jnp.maximum(tx1[j], px1), 0.0)
        iy = jnp.maximum(jnp.minimum(ty2[j], py2) - jnp.maximum(ty1[j], py1), 0.0)
        inter = ix * iy
        area_t = (tx2[j] - tx1[j]) * (ty2[j] - ty1[j])
        ov = inter / (area_t + area_p - inter)
        if j == 0:
            bto = ov
            bti = jnp.zeros((_R, 128), jnp.int32)
        else:
            upd = ov > bto
            bti = jnp.where(upd, jnp.int32(j), bti)
            bto = jnp.where(upd, ov, bto)
        m = jnp.max(ov)
        bpi.append(jnp.min(jnp.where(ov == m, fi, jnp.int32(2 ** 30))))

    for j in range(_NUM_OBJS):
        hit = fi == bpi[j]
        bto = jnp.where(hit, 2.0, bto)
        bti = jnp.where(hit, jnp.int32(j), bti)

    conf_t = jnp.where(bti == 0, tlab[0].astype(jnp.int32), 0)
    mx1 = jnp.where(bti == 0, tx1[0], 0.0)
    my1 = jnp.where(bti == 0, ty1[0], 0.0)
    mx2 = jnp.where(bti == 0, tx2[0], 0.0)
    my2 = jnp.where(bti == 0, ty2[0], 0.0)
    for j in range(1, _NUM_OBJS):
        sel = bti == j
        conf_t = jnp.where(sel, tlab[j].astype(jnp.int32), conf_t)
        mx1 = jnp.where(sel, tx1[j], mx1)
        my1 = jnp.where(sel, ty1[j], my1)
        mx2 = jnp.where(sel, tx2[j], mx2)
        my2 = jnp.where(sel, ty2[j], my2)
    conf_t = jnp.where(bto < _THRESHOLD, 0, conf_t)
    pos = conf_t > 0
    posf = pos.astype(jnp.float32)
    num_pos = jnp.sum(conf_t > 0, dtype=jnp.int32)

    g_cx = ((mx1 + mx2) * 0.5 - pcx) / (_VAR0 * pw)
    g_cy = ((my1 + my2) * 0.5 - pcy) / (_VAR0 * ph)
    g_w = jnp.log((mx2 - mx1) / pw) / _VAR1
    g_h = jnp.log((my2 - my1) / ph) / _VAR1
    loss_l = jnp.float32(0.0)
    for c, g in enumerate((g_cx, g_cy, g_w, g_h)):
        d = loc_ref[0, c] - g
        ad = jnp.abs(d)
        sl1 = jnp.where(ad < 1.0, 0.5 * d * d, ad - 0.5)
        loss_l = loss_l + jnp.sum(sl1 * posf)

    o0 = obj_ref[0, 0]
    o1 = obj_ref[0, 1]
    mo = jnp.maximum(o0, o1)
    lse_o = mo + jnp.log(jnp.exp(o0 - mo) + jnp.exp(o1 - mo))
    ce_obj = lse_o - jnp.where(pos, o1, o0)

    mc = conf_ref[0, 0]
    for c in range(1, _NUM_CLASSES):
        mc = jnp.maximum(mc, conf_ref[0, c])
    s = jnp.exp(conf_ref[0, 0] - mc)
    for c in range(1, _NUM_CLASSES):
        s = s + jnp.exp(conf_ref[0, c] - mc)
    lse_c = mc + jnp.log(s)
    c_sel = jnp.zeros((_R, 128), jnp.float32)
    for c in range(_NUM_CLASSES):
        c_sel = jnp.where(conf_t == c + 1, conf_ref[0, c], c_sel)
    ce_all = jnp.where(pos, lse_c + lse_o - o1 - c_sel, ce_obj)

    k = jnp.minimum(jnp.int32(_NEGPOS_RATIO) * num_pos,
                    jnp.int32(_NUM_PRIORS - 1))
    mined = jnp.where(pos, 0.0, ce_obj)
    tk = _topk_sum(mined, k)
    loss_c = jnp.sum(jnp.where(pos, ce_all, 0.0)) + tk
    loss_o = jnp.sum(jnp.where(pos, ce_obj, 0.0)) + tk

    lane = lax.broadcasted_iota(jnp.int32, (1, 8), 1)
    row = (jnp.where(lane == 0, loss_l, 0.0)
           + jnp.where(lane == 1, loss_c, 0.0)
           + jnp.where(lane == 2, loss_o, 0.0)
           + jnp.where(lane == 3, num_pos.astype(jnp.float32), 0.0))

    @pl.when(b == 0)
    def _init():
        out_ref[...] = row

    @pl.when(b > 0)
    def _acc():
        out_ref[...] = out_ref[...] + row


@jax.jit
def _run(loc_s, conf_s, obj_s, priors_s, targets):
    out = pl.pallas_call(
        _mbox_kernel,
        grid=(_BATCH,),
        in_specs=[
            pl.BlockSpec((4, _R, 128), lambda b: (0, 0, 0)),
            pl.BlockSpec((1, _NUM_OBJS, 5), lambda b: (b, 0, 0)),
            pl.BlockSpec((1, 4, _R, 128), lambda b: (b, 0, 0, 0)),
            pl.BlockSpec((1, _NUM_CLASSES, _R, 128), lambda b: (b, 0, 0, 0)),
            pl.BlockSpec((1, 2, _R, 128), lambda b: (b, 0, 0, 0)),
        ],
        out_specs=pl.BlockSpec((1, 8), lambda b: (0, 0)),
        out_shape=jax.ShapeDtypeStruct((1, 8), jnp.float32),
        compiler_params=pltpu.CompilerParams(
            dimension_semantics=("arbitrary",),
        ),
    )(priors_s, targets, loc_s, conf_s, obj_s)
    n = out[0, 3]
    return out[0, 0] / n, out[0, 1] / n, out[0, 2] / n


def kernel(loc_data, conf_data, obj_data, priors, targets):
    loc_s = loc_data.transpose(0, 2, 1).reshape(_BATCH, 4, _R, 128)
    conf_s = conf_data.transpose(0, 2, 1).reshape(_BATCH, _NUM_CLASSES, _R, 128)
    obj_s = obj_data.transpose(0, 2, 1).reshape(_BATCH, 2, _R, 128)
    priors_s = priors.T.reshape(4, _R, 128)
    return _run(loc_s, conf_s, obj_s, priors_s, targets)
